# baseline (speedup 1.0000x reference)
import jax
import jax.numpy as jnp
from jax import lax
from jax.experimental import pallas as pl
from jax.experimental.pallas import tpu as pltpu
from jax.experimental.pallas import tpu_sc as plsc

VOCAB = 1000000
D = 64
MAXLEN = 200
VDIM = 128
B = 1024
T = 200

NC = 2
NS = 16
NW = NC * NS
PER_W = B * T // NW

TOK_BLK = 128
N_FULL_BLK = VOCAB // TOK_BLK
TAIL_BLK = N_FULL_BLK
TAIL_W = VOCAB - N_FULL_BLK * TOK_BLK
N_BLK = N_FULL_BLK + 1
N_IT = 246

N_STREAMS = 50
STREAM = 128
GROUP_STREAMS = 5
GROUP = GROUP_STREAMS * STREAM
N_GROUPS = PER_W // GROUP


def _fmt_body(tableT_hbm, tail2_hbm, out_hbm, in_v0, in_v1, out_v, sem0, sem1):
    wid = lax.axis_index("s") * NC + lax.axis_index("c")

    def fire(i, buf, sem):
        blk = wid + i * NW

        @pl.when(blk < N_FULL_BLK)
        def _():
            pltpu.async_copy(
                tableT_hbm.at[:, pl.ds(blk * TOK_BLK, TOK_BLK)], buf, sem)

    def drain(i, buf, sem):
        blk = wid + i * NW

        @pl.when(blk < N_FULL_BLK)
        def _():
            pltpu.make_async_copy(
                tableT_hbm.at[:, pl.ds(0, TOK_BLK)], buf, sem).wait()

    def transpose_store(i, buf):
        blk = wid + i * NW

        @pl.when(blk < N_FULL_BLK)
        def _():
            for c0 in range(0, TOK_BLK, 16):
                rows = (c0 + lax.iota(jnp.int32, 16)) >> 1
                half = ((c0 + lax.iota(jnp.int32, 16)) & 1) * D
                for d in range(D):
                    plsc.store_scatter(
                        out_v, [rows, half + d], buf[d, pl.ds(c0, 16)])

        @pl.when(blk < N_FULL_BLK)
        def _():
            pltpu.sync_copy(
                out_v, out_hbm.at[pl.ds(blk * (TOK_BLK // 2), TOK_BLK // 2)])

    fire(0, in_v0, sem0)

    @pl.when(wid == 0)
    def _tail():
        pltpu.sync_copy(tail2_hbm, out_hbm.at[pl.ds(TAIL_BLK * (TOK_BLK // 2), TAIL_W // 2)])

    @pl.loop(0, N_IT // 2)
    def _ii(ii):
        i0 = ii * 2
        fire(i0 + 1, in_v1, sem1)
        drain(i0, in_v0, sem0)
        transpose_store(i0, in_v0)
        fire(i0 + 2, in_v0, sem0)
        drain(i0 + 1, in_v1, sem1)
        transpose_store(i0 + 1, in_v1)


def _sc_format(tableT, tail2):
    mesh = plsc.VectorSubcoreMesh(core_axis_name="c", subcore_axis_name="s")
    return pl.kernel(
        _fmt_body,
        out_type=jax.ShapeDtypeStruct((VOCAB // 2, 2 * D), jnp.float32),
        mesh=mesh,
        scratch_types=[
            pltpu.VMEM((D, TOK_BLK), jnp.float32),
            pltpu.VMEM((D, TOK_BLK), jnp.float32),
            pltpu.VMEM((TOK_BLK // 2, 2 * D), jnp.float32),
            pltpu.SemaphoreType.DMA,
            pltpu.SemaphoreType.DMA,
        ],
        compiler_params=pltpu.CompilerParams(needs_layout_passes=False),
    )(tableT, tail2)


def _gather_body(table_hbm, idx_hbm, out_hbm, idx_v, rows_v, sem):
    wid = lax.axis_index("s") * NC + lax.axis_index("c")
    base = wid * PER_W
    pltpu.sync_copy(idx_hbm.at[wid], idx_v)

    @pl.loop(0, N_GROUPS)
    def _group(g):
        copies = []
        for j in range(GROUP_STREAMS):
            copies.append(pltpu.async_copy(
                table_hbm.at[idx_v.at[g * GROUP_STREAMS + j]],
                rows_v.at[pl.ds(j * STREAM, STREAM)],
                sem,
            ))
        for c in copies:
            c.wait()
        pltpu.sync_copy(rows_v, out_hbm.at[pl.ds(base + g * GROUP, GROUP)])


def _sc_gather(table2, idx2):
    mesh = plsc.VectorSubcoreMesh(core_axis_name="c", subcore_axis_name="s")
    return pl.kernel(
        _gather_body,
        out_type=jax.ShapeDtypeStruct((B * T, 2 * D), jnp.float32),
        mesh=mesh,
        scratch_types=[
            pltpu.VMEM((N_STREAMS, STREAM), jnp.int32),
            pltpu.VMEM((GROUP, 2 * D), jnp.float32),
            pltpu.SemaphoreType.DMA,
        ],
    )(table2, idx2)


TBLK = 8


def _mm_body(vis_ref, w_ref, pos_ref, tmp_ref):
    for t in range(TBLK):
        v = vis_ref[:, t, :]
        p = lax.dot_general(
            w_ref[...], v, (((1,), (1,)), ((), ())),
            preferred_element_type=jnp.float32,
        )
        tmp_ref[t] = p + pos_ref[t][:, None]


def _tc_matmul(vis, w, pos):
    return pl.pallas_call(
        _mm_body,
        grid=(T // TBLK,),
        in_specs=[
            pl.BlockSpec((B, TBLK, VDIM), lambda i: (0, i, 0)),
            pl.BlockSpec((D, VDIM), lambda i: (0, 0)),
            pl.BlockSpec((TBLK, D), lambda i: (i, 0)),
        ],
        out_specs=pl.BlockSpec((TBLK, D, B), lambda i: (i, 0, 0)),
        out_shape=jax.ShapeDtypeStruct((T, D, B), jnp.float32),
    )(vis, w, pos)


def _fin_body(g_ref, tmp_ref, par_ref, gamma_ref, beta_ref, out_ref):
    g2 = g_ref[...]
    lo = g2[:, 0:D, :]
    hi = g2[:, D:2 * D, :]
    par = par_ref[...]
    x = jnp.where(par > 0.5, hi, lo) + tmp_ref[...]
    mean = jnp.mean(x, axis=1, keepdims=True)
    xc = x - mean
    var = jnp.mean(xc * xc, axis=1, keepdims=True)
    out_ref[...] = xc * lax.rsqrt(var + 1e-6) * gamma_ref[...] + beta_ref[...]


def _tc_final(g_t, tmp, par, gamma, beta):
    return pl.pallas_call(
        _fin_body,
        grid=(T // TBLK,),
        in_specs=[
            pl.BlockSpec((TBLK, 2 * D, B), lambda i: (i, 0, 0)),
            pl.BlockSpec((TBLK, D, B), lambda i: (i, 0, 0)),
            pl.BlockSpec((TBLK, 1, B), lambda i: (i, 0, 0)),
            pl.BlockSpec((1, D, 1), lambda i: (0, 0, 0)),
            pl.BlockSpec((1, D, 1), lambda i: (0, 0, 0)),
        ],
        out_specs=pl.BlockSpec((TBLK, D, B), lambda i: (i, 0, 0)),
        out_shape=jax.ShapeDtypeStruct((T, D, B), jnp.float32),
    )(g_t, tmp, par, gamma, beta)


def kernel(seq, visual_features, token_table, pos_table, W_visual, ln_gamma, ln_beta):
    seq_i = seq.astype(jnp.int32)
    idx2 = (seq_i >> 1).reshape(NW, N_STREAMS, STREAM)
    tail2 = token_table[N_FULL_BLK * TOK_BLK:].reshape(TAIL_W // 2, 2 * D)
    table2 = _sc_format(token_table.T, tail2)
    gathered2 = _sc_gather(table2, idx2).reshape(B, T, 2 * D)
    tmp = _tc_matmul(visual_features, W_visual, pos_table)
    g2_t = jnp.transpose(gathered2, (1, 2, 0))
    par_t = jnp.transpose((seq_i & 1).astype(jnp.float32), (1, 0)).reshape(T, 1, B)
    out_t = _tc_final(
        g2_t, tmp, par_t, ln_gamma.reshape(1, D, 1), ln_beta.reshape(1, D, 1)
    )
    return jnp.transpose(out_t, (2, 0, 1))


# speedup vs baseline: 1.8536x; 1.8536x over previous
import jax
import jax.numpy as jnp
from jax import lax
from jax.experimental import pallas as pl
from jax.experimental.pallas import tpu as pltpu
from jax.experimental.pallas import tpu_sc as plsc

VOCAB = 1000000
D = 64
MAXLEN = 200
VDIM = 128
B = 1024
T = 200

NC = 2
NS = 16
NW = NC * NS
PER_W = B * T // NW


N_STREAMS = 50
STREAM = 128
GROUP_STREAMS = 5
GROUP = GROUP_STREAMS * STREAM
N_GROUPS = PER_W // GROUP


def _sc_gather_body(table_hbm, idx_hbm, out_hbm, idx_v, rows_v, sem):
    wid = lax.axis_index("s") * NC + lax.axis_index("c")
    base = wid * PER_W
    pltpu.sync_copy(idx_hbm.at[wid], idx_v)

    @pl.loop(0, N_GROUPS)
    def _group(g):
        copies = []
        for j in range(GROUP_STREAMS):
            copies.append(pltpu.async_copy(
                table_hbm.at[idx_v.at[g * GROUP_STREAMS + j]],
                rows_v.at[pl.ds(j * STREAM, STREAM)],
                sem,
            ))
        for c in copies:
            c.wait()
        pltpu.sync_copy(rows_v, out_hbm.at[pl.ds(base + g * GROUP, GROUP)])


def _sc_gather(table, idx):
    mesh = plsc.VectorSubcoreMesh(core_axis_name="c", subcore_axis_name="s")
    return pl.kernel(
        _sc_gather_body,
        out_type=jax.ShapeDtypeStruct((B * T, D), jnp.float32),
        mesh=mesh,
        scratch_types=[
            pltpu.VMEM((N_STREAMS, STREAM), jnp.int32),
            pltpu.VMEM((GROUP, D), jnp.float32),
            pltpu.SemaphoreType.DMA,
        ],
        compiler_params=pltpu.CompilerParams(use_tc_tiling_on_sc=False),
    )(table, idx)


TBLK = 8


def _mm_body(vis_ref, w_ref, pos_ref, tmp_ref):
    for t in range(TBLK):
        v = vis_ref[:, t, :]
        p = lax.dot_general(
            w_ref[...], v, (((1,), (1,)), ((), ())),
            preferred_element_type=jnp.float32,
        )
        tmp_ref[t] = p + pos_ref[t][:, None]


def _tc_matmul(vis, w, pos):
    return pl.pallas_call(
        _mm_body,
        grid=(T // TBLK,),
        in_specs=[
            pl.BlockSpec((B, TBLK, VDIM), lambda i: (0, i, 0)),
            pl.BlockSpec((D, VDIM), lambda i: (0, 0)),
            pl.BlockSpec((TBLK, D), lambda i: (i, 0)),
        ],
        out_specs=pl.BlockSpec((TBLK, D, B), lambda i: (i, 0, 0)),
        out_shape=jax.ShapeDtypeStruct((T, D, B), jnp.float32),
    )(vis, w, pos)


def _fin_body(g_ref, tmp_ref, gamma_ref, beta_ref, out_ref):
    x = g_ref[...].reshape(TBLK, D, B) + tmp_ref[...]
    mean = jnp.mean(x, axis=1, keepdims=True)
    xc = x - mean
    var = jnp.mean(xc * xc, axis=1, keepdims=True)
    out_ref[...] = xc * lax.rsqrt(var + 1e-6) * gamma_ref[...] + beta_ref[...]


def _tc_final(g_t, tmp, gamma, beta):
    return pl.pallas_call(
        _fin_body,
        grid=(T // TBLK,),
        in_specs=[
            pl.BlockSpec((TBLK // 2, 2 * D, B), lambda i: (i, 0, 0)),
            pl.BlockSpec((TBLK, D, B), lambda i: (i, 0, 0)),
            pl.BlockSpec((1, D, 1), lambda i: (0, 0, 0)),
            pl.BlockSpec((1, D, 1), lambda i: (0, 0, 0)),
        ],
        out_specs=pl.BlockSpec((TBLK, D, B), lambda i: (i, 0, 0)),
        out_shape=jax.ShapeDtypeStruct((T, D, B), jnp.float32),
    )(g_t, tmp, gamma, beta)


def kernel(seq, visual_features, token_table, pos_table, W_visual, ln_gamma, ln_beta):
    idx = seq.astype(jnp.int32).reshape(NW, N_STREAMS, STREAM)
    gathered = _sc_gather(token_table, idx).reshape(B, T // 2, 2 * D)
    tmp = _tc_matmul(visual_features, W_visual, pos_table)
    g_t = jnp.transpose(gathered, (1, 2, 0))
    out_t = _tc_final(
        g_t, tmp, ln_gamma.reshape(1, D, 1), ln_beta.reshape(1, D, 1)
    )
    return jnp.transpose(out_t, (2, 0, 1))
